# baseline (speedup 1.0000x reference)
import functools

import jax
import jax.numpy as jnp
from jax import lax
from jax.experimental import pallas as pl
from jax.experimental.pallas import tpu as pltpu
from jax.experimental.pallas import tpu_sc as plsc

EMB_DIM = 128
BATCH = 4096
SEQ = 50
TOTAL = BATCH * SEQ
NUM_WORKERS = 32
RPW = TOTAL // NUM_WORKERS
SLAB = 200
DLEN = 100
DPS = SLAB // DLEN
NG = RPW // SLAB
NBUF = 4

_mesh = plsc.VectorSubcoreMesh(core_axis_name="c", subcore_axis_name="s")


@functools.partial(
    pl.kernel,
    mesh=_mesh,
    out_type=jax.ShapeDtypeStruct((TOTAL, EMB_DIM), jnp.float32),
    scratch_types=[
        pltpu.VMEM((NG * DPS, DLEN), jnp.int32),
        pltpu.VMEM((NBUF, SLAB, EMB_DIM), jnp.float32),
        pltpu.SemaphoreType.DMA,
        pltpu.SemaphoreType.DMA,
    ],
)
def _emb_lookup(idx_hbm, table_hbm, out_hbm, idx_v, rows_v, gsem, osem):
    wid = lax.axis_index("s") * 2 + lax.axis_index("c")
    row0 = wid * RPW
    pltpu.sync_copy(idx_hbm.at[wid], idx_v)

    def gfire(t, b):
        for d in range(DPS):
            pltpu.async_copy(
                table_hbm.at[idx_v.at[t * DPS + d]],
                rows_v.at[b].at[pl.ds(d * DLEN, DLEN)], gsem)

    def ofire(t, b):
        return pltpu.async_copy(
            rows_v.at[b], out_hbm.at[pl.ds(row0 + t * SLAB, SLAB)], osem)

    def gwait():
        for d in range(DPS):
            pltpu.make_async_copy(
                table_hbm.at[idx_v.at[0]],
                rows_v.at[0].at[pl.ds(0, DLEN)], gsem).wait()

    def owait():
        pltpu.make_async_copy(
            rows_v.at[0], out_hbm.at[pl.ds(row0, SLAB)], osem).wait()

    for b in range(NBUF - 1):
        gfire(b, b)
    gfire(NBUF - 1, NBUF - 1)
    gwait()
    ofire(0, 0)

    def body(t, carry):
        owait()
        gfire(t + NBUF - 1, lax.rem(t + NBUF - 1, NBUF))
        gwait()
        ofire(t, lax.rem(t, NBUF))
        return carry

    lax.fori_loop(1, NG - NBUF + 1, body, 0)

    for k in range(NBUF - 1):
        t = NG - NBUF + 1 + k
        gwait()
        ofire(t, t % NBUF)
    for _ in range(NBUF):
        owait()


@jax.jit
def kernel(x, table):
    idx = x.T.reshape(NUM_WORKERS, NG * DPS, DLEN).astype(jnp.int32)
    out = _emb_lookup(idx, table)
    return out.reshape(SEQ, BATCH, EMB_DIM).transpose(1, 0, 2)


# speedup vs baseline: 1.0118x; 1.0118x over previous
import functools

import jax
import jax.numpy as jnp
from jax import lax
from jax.experimental import pallas as pl
from jax.experimental.pallas import tpu as pltpu
from jax.experimental.pallas import tpu_sc as plsc

EMB_DIM = 128
BATCH = 4096
SEQ = 50
TOTAL = BATCH * SEQ
NUM_WORKERS = 32
RPW = TOTAL // NUM_WORKERS
SLAB = 256
DLEN = 128
DPS = SLAB // DLEN
NG = RPW // SLAB
NBUF = 3

_mesh = plsc.VectorSubcoreMesh(core_axis_name="c", subcore_axis_name="s")


@functools.partial(
    pl.kernel,
    mesh=_mesh,
    out_type=jax.ShapeDtypeStruct((TOTAL, EMB_DIM), jnp.float32),
    scratch_types=[
        pltpu.VMEM((NG * DPS, DLEN), jnp.int32),
        pltpu.VMEM((NBUF, SLAB, EMB_DIM), jnp.float32),
        pltpu.SemaphoreType.DMA,
        pltpu.SemaphoreType.DMA,
    ],
)
def _emb_lookup(idx_hbm, table_hbm, out_hbm, idx_v, rows_v, gsem, osem):
    wid = lax.axis_index("s") * 2 + lax.axis_index("c")
    row0 = wid * RPW
    pltpu.sync_copy(idx_hbm.at[wid], idx_v)

    def gfire(t, b):
        for d in range(DPS):
            pltpu.async_copy(
                table_hbm.at[idx_v.at[t * DPS + d]],
                rows_v.at[b].at[pl.ds(d * DLEN, DLEN)], gsem)

    def ofire(t, b):
        return pltpu.async_copy(
            rows_v.at[b], out_hbm.at[pl.ds(row0 + t * SLAB, SLAB)], osem)

    def gwait():
        for d in range(DPS):
            pltpu.make_async_copy(
                table_hbm.at[idx_v.at[0]],
                rows_v.at[0].at[pl.ds(0, DLEN)], gsem).wait()

    def owait():
        pltpu.make_async_copy(
            rows_v.at[0], out_hbm.at[pl.ds(row0, SLAB)], osem).wait()

    for b in range(NBUF - 1):
        gfire(b, b)
    gfire(NBUF - 1, NBUF - 1)
    gwait()
    ofire(0, 0)

    def body(t, carry):
        owait()
        gfire(t + NBUF - 1, lax.rem(t + NBUF - 1, NBUF))
        gwait()
        ofire(t, lax.rem(t, NBUF))
        return carry

    lax.fori_loop(1, NG - NBUF + 1, body, 0)

    for k in range(NBUF - 1):
        t = NG - NBUF + 1 + k
        gwait()
        ofire(t, t % NBUF)
    for _ in range(NBUF):
        owait()


@jax.jit
def kernel(x, table):
    idx = x.T.reshape(NUM_WORKERS, NG * DPS, DLEN).astype(jnp.int32)
    out = _emb_lookup(idx, table)
    return out.reshape(SEQ, BATCH, EMB_DIM).transpose(1, 0, 2)


# speedup vs baseline: 1.0126x; 1.0008x over previous
import functools

import jax
import jax.numpy as jnp
from jax import lax
from jax.experimental import pallas as pl
from jax.experimental.pallas import tpu as pltpu
from jax.experimental.pallas import tpu_sc as plsc

EMB_DIM = 128
BATCH = 4096
SEQ = 50
TOTAL = BATCH * SEQ
NUM_WORKERS = 32
RPW = TOTAL // NUM_WORKERS
SLAB = 128
DLEN = 128
DPS = SLAB // DLEN
NG = RPW // SLAB
NBUF = 6

_mesh = plsc.VectorSubcoreMesh(core_axis_name="c", subcore_axis_name="s")


@functools.partial(
    pl.kernel,
    mesh=_mesh,
    out_type=jax.ShapeDtypeStruct((TOTAL, EMB_DIM), jnp.float32),
    scratch_types=[
        pltpu.VMEM((NG * DPS, DLEN), jnp.int32),
        pltpu.VMEM((NBUF, SLAB, EMB_DIM), jnp.float32),
        pltpu.SemaphoreType.DMA,
        pltpu.SemaphoreType.DMA,
    ],
)
def _emb_lookup(idx_hbm, table_hbm, out_hbm, idx_v, rows_v, gsem, osem):
    wid = lax.axis_index("s") * 2 + lax.axis_index("c")
    row0 = wid * RPW
    pltpu.sync_copy(idx_hbm.at[wid], idx_v)

    def gfire(t, b):
        for d in range(DPS):
            pltpu.async_copy(
                table_hbm.at[idx_v.at[t * DPS + d]],
                rows_v.at[b].at[pl.ds(d * DLEN, DLEN)], gsem)

    def ofire(t, b):
        return pltpu.async_copy(
            rows_v.at[b], out_hbm.at[pl.ds(row0 + t * SLAB, SLAB)], osem)

    def gwait():
        for d in range(DPS):
            pltpu.make_async_copy(
                table_hbm.at[idx_v.at[0]],
                rows_v.at[0].at[pl.ds(0, DLEN)], gsem).wait()

    def owait():
        pltpu.make_async_copy(
            rows_v.at[0], out_hbm.at[pl.ds(row0, SLAB)], osem).wait()

    for b in range(NBUF - 1):
        gfire(b, b)
    gfire(NBUF - 1, NBUF - 1)
    gwait()
    ofire(0, 0)

    def body(t, carry):
        owait()
        gfire(t + NBUF - 1, lax.rem(t + NBUF - 1, NBUF))
        gwait()
        ofire(t, lax.rem(t, NBUF))
        return carry

    lax.fori_loop(1, NG - NBUF + 1, body, 0)

    for k in range(NBUF - 1):
        t = NG - NBUF + 1 + k
        gwait()
        ofire(t, t % NBUF)
    for _ in range(NBUF):
        owait()


@jax.jit
def kernel(x, table):
    idx = x.T.reshape(NUM_WORKERS, NG * DPS, DLEN).astype(jnp.int32)
    out = _emb_lookup(idx, table)
    return out.reshape(SEQ, BATCH, EMB_DIM).transpose(1, 0, 2)


# speedup vs baseline: 1.0134x; 1.0007x over previous
import functools

import jax
import jax.numpy as jnp
from jax import lax
from jax.experimental import pallas as pl
from jax.experimental.pallas import tpu as pltpu
from jax.experimental.pallas import tpu_sc as plsc

EMB_DIM = 128
BATCH = 4096
SEQ = 50
TOTAL = BATCH * SEQ
NUM_WORKERS = 32
RPW = TOTAL // NUM_WORKERS
SLAB = 128
DLEN = 128
DPS = SLAB // DLEN
NG = RPW // SLAB
NBUF = 6

_mesh = plsc.VectorSubcoreMesh(core_axis_name="c", subcore_axis_name="s")


@functools.partial(
    pl.kernel,
    mesh=_mesh,
    out_type=jax.ShapeDtypeStruct((TOTAL, EMB_DIM), jnp.float32),
    scratch_types=[
        pltpu.VMEM((NG * DPS, DLEN), jnp.int32),
        pltpu.VMEM((NBUF, SLAB, EMB_DIM), jnp.float32),
        pltpu.SemaphoreType.DMA,
        pltpu.SemaphoreType.DMA,
    ],
)
def _emb_lookup(idx_hbm, table_hbm, out_hbm, idx_v, rows_v, gsem, osem):
    wid = lax.axis_index("s") * 2 + lax.axis_index("c")
    row0 = wid * RPW
    pltpu.sync_copy(idx_hbm.at[wid], idx_v)

    def gfire(t, b):
        for d in range(DPS):
            pltpu.async_copy(
                table_hbm.at[idx_v.at[t * DPS + d]],
                rows_v.at[b].at[pl.ds(d * DLEN, DLEN)], gsem)

    def ofire(t, b):
        return pltpu.async_copy(
            rows_v.at[b], out_hbm.at[pl.ds(row0 + t * SLAB, SLAB)], osem)

    def gwait():
        for d in range(DPS):
            pltpu.make_async_copy(
                table_hbm.at[idx_v.at[0]],
                rows_v.at[0].at[pl.ds(0, DLEN)], gsem).wait()

    def owait():
        pltpu.make_async_copy(
            rows_v.at[0], out_hbm.at[pl.ds(row0, SLAB)], osem).wait()

    for b in range(NBUF):
        gfire(b, b)
    gwait()
    ofire(0, 0)

    def body(t, carry):
        owait()
        gfire(t + NBUF - 1, lax.rem(t + NBUF - 1, NBUF))
        gwait()
        ofire(t, lax.rem(t, NBUF))
        return carry

    lax.fori_loop(1, NG - NBUF + 1, body, 0)

    for k in range(NBUF - 1):
        t = NG - NBUF + 1 + k
        gwait()
        ofire(t, t % NBUF)
    for _ in range(NBUF):
        owait()


@jax.jit
def kernel(x, table):
    idx = x.T.reshape(NUM_WORKERS, NG * DPS, DLEN).astype(jnp.int32)
    out = _emb_lookup(idx, table)
    return out.reshape(SEQ, BATCH, EMB_DIM).transpose(1, 0, 2)
